# Initial kernel scaffold; baseline (speedup 1.0000x reference)
import jax
import jax.numpy as jnp
from jax.experimental import pallas as pl


def kernel(x, edge_index, W_self1, W_neigh1, b1, W_self2, W_neigh2, b2, W_head, b_head):
    raise NotImplementedError("write your pallas kernel here")


# speedup vs baseline: 5.6509x; 5.6509x over previous
import functools

import jax
import jax.numpy as jnp
from jax import lax
from jax.experimental import pallas as pl
from jax.experimental.pallas import tpu as pltpu
from jax.experimental.pallas import tpu_sc as plsc

N = 10000
E = 320000
D = 128
H = 64
O = 2

NC = 2
NS = 16
NW = NC * NS

LANES = 128
KJ = 8
R = 10240
ROWS_PER_TILE = R // NS
E_PAD = 327680
IDX_ROWS = E_PAD // LANES
ROWS_PER_WORKER = IDX_ROWS // NW
STEPS = ROWS_PER_WORKER // KJ

BN = 1000
GRID = N // BN


def _make_sc_aggregate(with_deg: bool):
    mesh = plsc.VectorSubcoreMesh(core_axis_name="c", subcore_axis_name="s",
                                  num_cores=NC, num_subcores=NS)

    out_type = [jax.ShapeDtypeStruct((NC, R, H), jnp.float32)]
    scratch = [
        pltpu.VMEM((KJ, LANES), jnp.int32),
        pltpu.VMEM((KJ, LANES), jnp.int32),
        pltpu.VMEM((KJ, LANES, H), jnp.float32),
        pltpu.VMEM_SHARED((R, H), jnp.float32),
        pltpu.SemaphoreType.DMA,
    ]
    if with_deg:
        out_type.append(jax.ShapeDtypeStruct((NC, R, 16), jnp.float32))
        scratch += [
            pltpu.VMEM((LANES, 16), jnp.float32),
            pltpu.VMEM_SHARED((R, 16), jnp.float32)
        ]

    def body(p_hbm, src_hbm, dst_hbm, zeros_hbm, zeros16_hbm, ones_hbm,
             *rest):
        if with_deg:
            (agg_out, deg_out, src_v, dst_v, rows_v, acc_sh, sem,
             ones_v, deg_sh) = rest
        else:
            agg_out, src_v, dst_v, rows_v, acc_sh, sem = rest
        cid = lax.axis_index("c")
        sid = lax.axis_index("s")
        wid = cid * NS + sid
        r0 = sid * ROWS_PER_TILE

        pltpu.sync_copy(zeros_hbm.at[pl.ds(r0, ROWS_PER_TILE)],
                        acc_sh.at[pl.ds(r0, ROWS_PER_TILE)])
        if with_deg:
            pltpu.sync_copy(zeros16_hbm.at[pl.ds(r0, ROWS_PER_TILE)],
                            deg_sh.at[pl.ds(r0, ROWS_PER_TILE)])
            pltpu.sync_copy(ones_hbm, ones_v)
        plsc.subcore_barrier()

        base = wid * ROWS_PER_WORKER

        def step(t, carry):
            row = base + t * KJ
            pltpu.sync_copy(src_hbm.at[pl.ds(row, KJ)], src_v)
            pltpu.sync_copy(dst_hbm.at[pl.ds(row, KJ)], dst_v)
            copies = [
                pltpu.async_copy(p_hbm.at[src_v.at[j]], rows_v.at[j], sem)
                for j in range(KJ)
            ]
            for c in copies:
                c.wait()
            for j in range(KJ):
                pltpu.sync_copy(rows_v.at[j], acc_sh.at[dst_v.at[j]],
                                add=True)
            if with_deg:
                for j in range(KJ):
                    pltpu.sync_copy(ones_v, deg_sh.at[dst_v.at[j]],
                                    add=True)
            return carry

        lax.fori_loop(0, STEPS, step, 0)
        plsc.subcore_barrier()

        pltpu.sync_copy(acc_sh.at[pl.ds(r0, ROWS_PER_TILE)],
                        agg_out.at[cid, pl.ds(r0, ROWS_PER_TILE)])
        if with_deg:
            pltpu.sync_copy(deg_sh.at[pl.ds(r0, ROWS_PER_TILE)],
                            deg_out.at[cid, pl.ds(r0, ROWS_PER_TILE)])

    return pl.kernel(
        body, out_type=tuple(out_type), mesh=mesh,
        scratch_types=tuple(scratch),
        compiler_params=pltpu.CompilerParams(use_tc_tiling_on_sc=False))


_sc_agg_deg = _make_sc_aggregate(with_deg=True)
_sc_agg = _make_sc_aggregate(with_deg=False)


def _proj2_body(x_ref, wa_ref, wb_ref, oa_ref, ob_ref):
    xb = x_ref[...]
    oa_ref[...] = jnp.dot(xb, wa_ref[...], preferred_element_type=jnp.float32)
    ob_ref[...] = jnp.dot(xb, wb_ref[...], preferred_element_type=jnp.float32)


def _proj2(x, wa, wb):
    d_in = x.shape[1]
    h_out = wa.shape[1]
    return pl.pallas_call(
        _proj2_body,
        grid=(GRID,),
        in_specs=[
            pl.BlockSpec((BN, d_in), lambda i: (i, 0)),
            pl.BlockSpec((d_in, h_out), lambda i: (0, 0)),
            pl.BlockSpec((d_in, h_out), lambda i: (0, 0)),
        ],
        out_specs=[
            pl.BlockSpec((BN, h_out), lambda i: (i, 0)),
            pl.BlockSpec((BN, h_out), lambda i: (i, 0)),
        ],
        out_shape=[
            jax.ShapeDtypeStruct((N, h_out), jnp.float32),
            jax.ShapeDtypeStruct((N, h_out), jnp.float32),
        ],
    )(x, wa, wb)


def _layer2_body(s_ref, aggp_ref, degp_ref, b_ref, wa_ref, wb_ref,
                 oa_ref, ob_ref):
    agg = aggp_ref[0] + aggp_ref[1]
    deg = (degp_ref[0] + degp_ref[1])[:, 0:1]
    h = jnp.maximum(s_ref[...] + agg / jnp.maximum(deg, 1.0) + b_ref[...],
                    0.0)
    oa_ref[...] = jnp.dot(h, wa_ref[...], preferred_element_type=jnp.float32)
    ob_ref[...] = jnp.dot(h, wb_ref[...], preferred_element_type=jnp.float32)


def _layer2(s, aggp, degp, b, wa, wb):
    return pl.pallas_call(
        _layer2_body,
        grid=(GRID,),
        in_specs=[
            pl.BlockSpec((BN, H), lambda i: (i, 0)),
            pl.BlockSpec((NC, BN, H), lambda i: (0, i, 0)),
            pl.BlockSpec((NC, BN, 16), lambda i: (0, i, 0)),
            pl.BlockSpec((1, H), lambda i: (0, 0)),
            pl.BlockSpec((H, H), lambda i: (0, 0)),
            pl.BlockSpec((H, H), lambda i: (0, 0)),
        ],
        out_specs=[
            pl.BlockSpec((BN, H), lambda i: (i, 0)),
            pl.BlockSpec((BN, H), lambda i: (i, 0)),
        ],
        out_shape=[
            jax.ShapeDtypeStruct((N, H), jnp.float32),
            jax.ShapeDtypeStruct((N, H), jnp.float32),
        ],
    )(s, aggp, degp, b, wa, wb)


def _head_body(s_ref, aggp_ref, degp_ref, b_ref, wh_ref, bh_ref, o_ref):
    agg = aggp_ref[0] + aggp_ref[1]
    deg = (degp_ref[0] + degp_ref[1])[:, 0:1]
    h = jnp.maximum(s_ref[...] + agg / jnp.maximum(deg, 1.0) + b_ref[...],
                    0.0)
    o_ref[...] = (jnp.dot(h, wh_ref[...], preferred_element_type=jnp.float32)
                  + bh_ref[...])


def _head(s, aggp, degp, b, wh_pad, bh_pad):
    return pl.pallas_call(
        _head_body,
        grid=(GRID,),
        in_specs=[
            pl.BlockSpec((BN, H), lambda i: (i, 0)),
            pl.BlockSpec((NC, BN, H), lambda i: (0, i, 0)),
            pl.BlockSpec((NC, BN, 16), lambda i: (0, i, 0)),
            pl.BlockSpec((1, H), lambda i: (0, 0)),
            pl.BlockSpec((H, 128), lambda i: (0, 0)),
            pl.BlockSpec((1, 128), lambda i: (0, 0)),
        ],
        out_specs=pl.BlockSpec((BN, 128), lambda i: (i, 0)),
        out_shape=jax.ShapeDtypeStruct((N, 128), jnp.float32),
    )(s, aggp, degp, b, wh_pad, bh_pad)


def kernel(x, edge_index, W_self1, W_neigh1, b1, W_self2, W_neigh2, b2,
           W_head, b_head):
    src = edge_index[0].astype(jnp.int32)
    dst = edge_index[1].astype(jnp.int32)
    pad = E_PAD - E
    src_p = jnp.concatenate([src, jnp.zeros((pad,), jnp.int32)])
    dst_p = jnp.concatenate([dst, jnp.full((pad,), N, jnp.int32)])
    src_p = src_p.reshape(IDX_ROWS, LANES)
    dst_p = dst_p.reshape(IDX_ROWS, LANES)

    zeros64 = jnp.zeros((R, H), jnp.float32)
    zeros16 = jnp.zeros((R, 16), jnp.float32)
    ones16 = jnp.ones((LANES, 16), jnp.float32)

    s1, p1 = _proj2(x, W_self1, W_neigh1)
    aggp1, degp = _sc_agg_deg(p1, src_p, dst_p, zeros64, zeros16, ones16)

    s2, p2 = _layer2(s1, aggp1, degp, b1.reshape(1, H), W_self2, W_neigh2)

    (aggp2,) = _sc_agg(p2, src_p, dst_p, zeros64, zeros16, ones16)

    wh_pad = jnp.zeros((H, 128), jnp.float32).at[:, :O].set(W_head)
    bh_pad = jnp.zeros((1, 128), jnp.float32).at[:, :O].set(
        b_head.reshape(1, O))
    out_pad = _head(s2, aggp2, degp, b2.reshape(1, H), wh_pad, bh_pad)
    return out_pad[:, :O]


# speedup vs baseline: 5.6666x; 1.0028x over previous
import functools

import jax
import jax.numpy as jnp
from jax import lax
from jax.experimental import pallas as pl
from jax.experimental.pallas import tpu as pltpu
from jax.experimental.pallas import tpu_sc as plsc

N = 10000
E = 320000
D = 128
H = 64
O = 2
W1 = H + 16

NC = 2
NS = 16
NW = NC * NS

LANES = 128
KJ = 4
KH = KJ // 2
R = 10240
ROWS_PER_TILE = R // NS
E_PAD = 327680
IDX_ROWS = E_PAD // LANES
ROWS_PER_WORKER = IDX_ROWS // NW
STEPS = ROWS_PER_WORKER // KJ

BN = 1000
GRID = N // BN


def _make_sc_aggregate(width):
    mesh = plsc.VectorSubcoreMesh(core_axis_name="c", subcore_axis_name="s",
                                  num_cores=NC, num_subcores=NS)

    def body(p_hbm, src_hbm, dst_hbm, agg_out,
             src_v, dst_v, rows_v, acc_sh, sem_g, sem_s, sem_i):
        cid = lax.axis_index("c")
        sid = lax.axis_index("s")
        wid = cid * NS + sid
        r0 = sid * ROWS_PER_TILE
        base = wid * ROWS_PER_WORKER

        ci0 = pltpu.async_copy(src_hbm.at[pl.ds(base, ROWS_PER_WORKER)],
                               src_v, sem_i)
        ci1 = pltpu.async_copy(dst_hbm.at[pl.ds(base, ROWS_PER_WORKER)],
                               dst_v, sem_i)

        def zstore(i, carry):
            for k in range(width // 16):
                rows_v[0, i, pl.ds(k * 16, 16)] = jnp.zeros((16,),
                                                            jnp.float32)
            return carry

        lax.fori_loop(0, LANES, zstore, 0)
        for q in range(ROWS_PER_TILE // LANES):
            pltpu.sync_copy(rows_v.at[0],
                            acc_sh.at[pl.ds(r0 + q * LANES, LANES)])
        ci0.wait()
        ci1.wait()
        plsc.subcore_barrier()

        def step(t, carry):
            row = t * KJ
            g0 = [pltpu.async_copy(p_hbm.at[src_v.at[row + j]],
                                   rows_v.at[j], sem_g)
                  for j in range(KH)]
            for h in g0:
                h.wait()
            s0 = [pltpu.async_copy(rows_v.at[j],
                                   acc_sh.at[dst_v.at[row + j]],
                                   sem_s, add=True)
                  for j in range(KH)]
            g1 = [pltpu.async_copy(p_hbm.at[src_v.at[row + KH + j]],
                                   rows_v.at[KH + j], sem_g)
                  for j in range(KH)]
            for h in g1:
                h.wait()
            s1 = [pltpu.async_copy(rows_v.at[KH + j],
                                   acc_sh.at[dst_v.at[row + KH + j]],
                                   sem_s, add=True)
                  for j in range(KH)]
            for h in s0 + s1:
                h.wait()
            return carry

        lax.fori_loop(0, STEPS, step, 0)
        plsc.subcore_barrier()

        pltpu.sync_copy(acc_sh.at[pl.ds(r0, ROWS_PER_TILE)],
                        agg_out.at[cid, pl.ds(r0, ROWS_PER_TILE)])

    return pl.kernel(
        body,
        out_type=jax.ShapeDtypeStruct((NC, R, width), jnp.float32),
        mesh=mesh,
        scratch_types=(
            pltpu.VMEM((ROWS_PER_WORKER, LANES), jnp.int32),
            pltpu.VMEM((ROWS_PER_WORKER, LANES), jnp.int32),
            pltpu.VMEM((KJ, LANES, width), jnp.float32),
            pltpu.VMEM_SHARED((R, width), jnp.float32),
            pltpu.SemaphoreType.DMA,
            pltpu.SemaphoreType.DMA,
            pltpu.SemaphoreType.DMA,
        ),
        compiler_params=pltpu.CompilerParams(use_tc_tiling_on_sc=False))


_sc_agg_w1 = _make_sc_aggregate(W1)
_sc_agg_w2 = _make_sc_aggregate(H)


def _proj1_body(x_ref, wa_ref, wb_ref, oa_ref, ob_ref):
    xb = x_ref[...]
    oa_ref[...] = jnp.dot(xb, wa_ref[...], preferred_element_type=jnp.float32)
    pb = jnp.dot(xb, wb_ref[...], preferred_element_type=jnp.float32)
    ob_ref[...] = jnp.concatenate(
        [pb, jnp.ones((BN, W1 - H), jnp.float32)], axis=1)


def _proj1(x, wa, wb):
    return pl.pallas_call(
        _proj1_body,
        grid=(GRID,),
        in_specs=[
            pl.BlockSpec((BN, D), lambda i: (i, 0)),
            pl.BlockSpec((D, H), lambda i: (0, 0)),
            pl.BlockSpec((D, H), lambda i: (0, 0)),
        ],
        out_specs=[
            pl.BlockSpec((BN, H), lambda i: (i, 0)),
            pl.BlockSpec((BN, W1), lambda i: (i, 0)),
        ],
        out_shape=[
            jax.ShapeDtypeStruct((N, H), jnp.float32),
            jax.ShapeDtypeStruct((N, W1), jnp.float32),
        ],
    )(x, wa, wb)


def _layer2_body(s_ref, aggp_ref, b_ref, wa_ref, wb_ref,
                 oa_ref, ob_ref, od_ref):
    comb = aggp_ref[0] + aggp_ref[1]
    deg = comb[:, H:H + 1]
    h = jnp.maximum(
        s_ref[...] + comb[:, :H] / jnp.maximum(deg, 1.0) + b_ref[...], 0.0)
    oa_ref[...] = jnp.dot(h, wa_ref[...], preferred_element_type=jnp.float32)
    ob_ref[...] = jnp.dot(h, wb_ref[...], preferred_element_type=jnp.float32)
    od_ref[...] = comb[:, H:]


def _layer2(s, aggp, b, wa, wb):
    return pl.pallas_call(
        _layer2_body,
        grid=(GRID,),
        in_specs=[
            pl.BlockSpec((BN, H), lambda i: (i, 0)),
            pl.BlockSpec((NC, BN, W1), lambda i: (0, i, 0)),
            pl.BlockSpec((1, H), lambda i: (0, 0)),
            pl.BlockSpec((H, H), lambda i: (0, 0)),
            pl.BlockSpec((H, H), lambda i: (0, 0)),
        ],
        out_specs=[
            pl.BlockSpec((BN, H), lambda i: (i, 0)),
            pl.BlockSpec((BN, H), lambda i: (i, 0)),
            pl.BlockSpec((BN, W1 - H), lambda i: (i, 0)),
        ],
        out_shape=[
            jax.ShapeDtypeStruct((N, H), jnp.float32),
            jax.ShapeDtypeStruct((N, H), jnp.float32),
            jax.ShapeDtypeStruct((N, W1 - H), jnp.float32),
        ],
    )(s, aggp, b, wa, wb)


def _head_body(s_ref, aggp_ref, deg_ref, b_ref, wh_ref, bh_ref, o_ref):
    agg = aggp_ref[0] + aggp_ref[1]
    deg = deg_ref[:, 0:1]
    h = jnp.maximum(
        s_ref[...] + agg / jnp.maximum(deg, 1.0) + b_ref[...], 0.0)
    o_ref[...] = (jnp.dot(h, wh_ref[...], preferred_element_type=jnp.float32)
                  + bh_ref[...])


def _head(s, aggp, deg, b, wh_pad, bh_pad):
    return pl.pallas_call(
        _head_body,
        grid=(GRID,),
        in_specs=[
            pl.BlockSpec((BN, H), lambda i: (i, 0)),
            pl.BlockSpec((NC, BN, H), lambda i: (0, i, 0)),
            pl.BlockSpec((BN, W1 - H), lambda i: (i, 0)),
            pl.BlockSpec((1, H), lambda i: (0, 0)),
            pl.BlockSpec((H, 128), lambda i: (0, 0)),
            pl.BlockSpec((1, 128), lambda i: (0, 0)),
        ],
        out_specs=pl.BlockSpec((BN, 128), lambda i: (i, 0)),
        out_shape=jax.ShapeDtypeStruct((N, 128), jnp.float32),
    )(s, aggp, deg, b, wh_pad, bh_pad)


def kernel(x, edge_index, W_self1, W_neigh1, b1, W_self2, W_neigh2, b2,
           W_head, b_head):
    src = edge_index[0].astype(jnp.int32)
    dst = edge_index[1].astype(jnp.int32)
    pad = E_PAD - E
    src_p = jnp.concatenate([src, jnp.zeros((pad,), jnp.int32)])
    dst_p = jnp.concatenate([dst, jnp.full((pad,), N, jnp.int32)])
    src_p = src_p.reshape(IDX_ROWS, LANES)
    dst_p = dst_p.reshape(IDX_ROWS, LANES)

    s1, p1 = _proj1(x, W_self1, W_neigh1)
    aggp1 = _sc_agg_w1(p1, src_p, dst_p)

    s2, p2, deg = _layer2(s1, aggp1, b1.reshape(1, H), W_self2, W_neigh2)

    aggp2 = _sc_agg_w2(p2, src_p, dst_p)

    wh_pad = jnp.zeros((H, 128), jnp.float32).at[:, :O].set(W_head)
    bh_pad = jnp.zeros((1, 128), jnp.float32).at[:, :O].set(
        b_head.reshape(1, O))
    out_pad = _head(s2, aggp2, deg, b2.reshape(1, H), wh_pad, bh_pad)
    return out_pad[:, :O]
